# Initial kernel scaffold; baseline (speedup 1.0000x reference)
import jax
import jax.numpy as jnp
from jax.experimental import pallas as pl


def kernel(extended_pitch, global_detuning, embedding_weight):
    raise NotImplementedError("write your pallas kernel here")


# speedup vs baseline: 186.5271x; 186.5271x over previous
import math

import jax
import jax.numpy as jnp
from jax import lax
from jax.experimental import pallas as pl
from jax.experimental.pallas import tpu as pltpu
from jax.experimental.pallas import tpu_sc as plsc

_B, _T = 32, 32768
_N = _B * _T
_NC, _NS = 2, 16
_NW = _NC * _NS
_PER = _N // _NW
_V = 16

_MAGIC = 12582912.0
_A = math.log(2.0) / 12.0
_BIAS = math.log(440.0) - 69.0 * _A


def _detune_body(p_hbm, d_hbm, w_hbm, out_hbm, p_v, d_v, o_v, w_v):
    wid = lax.axis_index("s") * _NC + lax.axis_index("c")
    base = wid * _PER
    pltpu.sync_copy(w_hbm, w_v)
    pltpu.sync_copy(p_hbm.at[pl.ds(base, _PER)], p_v)
    pltpu.sync_copy(d_hbm.at[pl.ds(base, _PER)], d_v)

    def step(i, carry):
        sl = pl.ds(i * _V, _V)
        p = p_v[sl]
        d = d_v[sl]
        r = (p + _MAGIC) - _MAGIC
        r = jnp.minimum(jnp.maximum(r, 0.0), 127.0)
        idx = r.astype(jnp.int32)
        t = plsc.load_gather(w_v, [idx])
        o_v[sl] = jnp.exp((p + d + t) * _A + _BIAS)
        return carry

    lax.fori_loop(0, _PER // _V, step, 0)
    pltpu.sync_copy(o_v, out_hbm.at[pl.ds(base, _PER)])


def kernel(extended_pitch, global_detuning, embedding_weight):
    p = extended_pitch.reshape(_N)
    d = global_detuning.reshape(_N)
    w = embedding_weight.reshape(128)
    mesh = plsc.VectorSubcoreMesh(core_axis_name="c", subcore_axis_name="s")
    f = pl.kernel(
        _detune_body,
        out_type=jax.ShapeDtypeStruct((_N,), jnp.float32),
        mesh=mesh,
        compiler_params=pltpu.CompilerParams(needs_layout_passes=False),
        scratch_types=[
            pltpu.VMEM((_PER,), jnp.float32),
            pltpu.VMEM((_PER,), jnp.float32),
            pltpu.VMEM((_PER,), jnp.float32),
            pltpu.VMEM((128,), jnp.float32),
        ],
    )
    out = f(p, d, w)
    return out.reshape(_B, _T, 1)


# speedup vs baseline: 343.7264x; 1.8428x over previous
import math

import jax
import jax.numpy as jnp
from jax import lax
from jax.experimental import pallas as pl
from jax.experimental.pallas import tpu as pltpu
from jax.experimental.pallas import tpu_sc as plsc

_B, _T = 32, 32768
_N = _B * _T
_NC, _NS = 2, 16
_NW = _NC * _NS
_PER = _N // _NW
_V = 16

_MAGIC = 12582912.0
_A = math.log(2.0) / 12.0
_BIAS = math.log(440.0) - 69.0 * _A


def _detune_body(p_hbm, d_hbm, w_hbm, out_hbm, p_v, d_v, o_v, w_v):
    wid = lax.axis_index("s") * _NC + lax.axis_index("c")
    base = wid * _PER
    pltpu.sync_copy(w_hbm, w_v)
    pltpu.sync_copy(p_hbm.at[pl.ds(base, _PER)], p_v)
    pltpu.sync_copy(d_hbm.at[pl.ds(base, _PER)], d_v)

    @plsc.parallel_loop(0, _PER, step=_V, unroll=8)
    def _step(i):
        sl = pl.ds(i, _V)
        p = p_v[sl]
        d = d_v[sl]
        r = (p + _MAGIC) - _MAGIC
        r = jnp.minimum(jnp.maximum(r, 0.0), 127.0)
        idx = r.astype(jnp.int32)
        t = plsc.load_gather(w_v, [idx])
        o_v[sl] = jnp.exp((p + d + t) * _A + _BIAS)
    pltpu.sync_copy(o_v, out_hbm.at[pl.ds(base, _PER)])


def kernel(extended_pitch, global_detuning, embedding_weight):
    p = extended_pitch.reshape(_N)
    d = global_detuning.reshape(_N)
    w = embedding_weight.reshape(128)
    mesh = plsc.VectorSubcoreMesh(core_axis_name="c", subcore_axis_name="s")
    f = pl.kernel(
        _detune_body,
        out_type=jax.ShapeDtypeStruct((_N,), jnp.float32),
        mesh=mesh,
        compiler_params=pltpu.CompilerParams(needs_layout_passes=False),
        scratch_types=[
            pltpu.VMEM((_PER,), jnp.float32),
            pltpu.VMEM((_PER,), jnp.float32),
            pltpu.VMEM((_PER,), jnp.float32),
            pltpu.VMEM((128,), jnp.float32),
        ],
    )
    out = f(p, d, w)
    return out.reshape(_B, _T, 1)
